# baseline (speedup 1.0000x reference)
import functools

import jax
import jax.numpy as jnp
from jax import lax
from jax.experimental import pallas as pl
from jax.experimental.pallas import tpu as pltpu
from jax.experimental.pallas import tpu_sc as plsc

N_FIELDS = 26
VOCAB = 100000
EMBED = 32
BATCH = 16384

NC, NS, L = 2, 16, 16
NW = NC * NS
TOTAL = BATCH * N_FIELDS
PER_W = TOTAL // NW
CHUNK = 1024
N_CH = PER_W // CHUNK
NBUF = 3

PLANES = N_FIELDS * EMBED
VFULL = VOCAB // 128
VREM = VOCAB - VFULL * 128
G_ROWS = N_FIELDS * VOCAB * EMBED // 128
FULL_BLOCKS = N_FIELDS * VFULL

_mesh = plsc.VectorSubcoreMesh(
    core_axis_name="c", subcore_axis_name="s", num_cores=NC, num_subcores=NS
)


@functools.partial(
    pl.kernel,
    out_type=jax.ShapeDtypeStruct((G_ROWS * 128,), jnp.float32),
    mesh=_mesh,
    compiler_params=pltpu.CompilerParams(needs_layout_passes=False),
    scratch_types=[
        pltpu.VMEM((EMBED, 513), jnp.float32),
        pltpu.VMEM((EMBED, 513), jnp.float32),
        pltpu.VMEM((128 * 128,), jnp.float32),
        pltpu.VMEM((128 * 128,), jnp.float32),
        [pltpu.SemaphoreType.DMA] * 2,
        [pltpu.SemaphoreType.DMA] * 2,
    ],
)
def _sc_detile(tt_hbm, rem_hbm, g_hbm, in_a, in_b, out_a, out_b, i_sems, o_sems):
    wid = lax.axis_index("s") * NC + lax.axis_index("c")
    ins = [in_a, in_b]
    outs = [out_a, out_b]

    W4 = 4
    VF4 = VFULL // W4
    SB_FULL = N_FIELDS * VF4
    NSB = SB_FULL // NW
    TAIL = SB_FULL - NSB * NW
    GPF = VOCAB * EMBED // 128

    def fc(i):
        s = i * NW + wid
        return s // VF4, s % VF4

    def start_in(i, k):
        f, c4 = fc(i)
        pltpu.async_copy(
            tt_hbm.at[pl.ds(f * EMBED, EMBED), pl.ds(c4 * 512, 512)],
            ins[k].at[pl.ds(0, EMBED), pl.ds(0, 512)],
            i_sems[k],
        )

    def start_out(i, k):
        f, c4 = fc(i)
        g0 = (f * GPF + c4 * 128) * 128
        return pltpu.async_copy(
            outs[k], g_hbm.at[pl.ds(g0, 128 * 128)], o_sems[k]
        )

    def drain_in(k):
        pltpu.make_async_copy(
            tt_hbm.at[pl.ds(0, EMBED), pl.ds(0, 512)],
            ins[k].at[pl.ds(0, EMBED), pl.ds(0, 512)],
            i_sems[k],
        ).wait()

    def drain_out(k):
        pltpu.make_async_copy(
            outs[k], g_hbm.at[pl.ds(0, 128 * 128)], o_sems[k]
        ).wait()

    iota = lax.iota(jnp.int32, L)

    def compute(k, nv=512):
        def vbody(v, carry):
            cidx = jnp.broadcast_to(v, (L,))
            x0 = plsc.load_gather(ins[k], [iota, cidx])
            x1 = plsc.load_gather(ins[k], [iota + 16, cidx])
            outs[k][pl.ds(v * 32, L)] = x0
            outs[k][pl.ds(v * 32 + 16, L)] = x1
            return carry

        lax.fori_loop(0, nv, vbody, 0)

    start_in(0, 0)
    start_in(1, 1)
    for k in (0, 1):
        drain_in(k)
        compute(k)
        start_in(2 + k, k)
        start_out(k, k)

    def pair(j, carry):
        for k in (0, 1):
            b = 2 * j + k
            drain_in(k)
            drain_out(k)
            compute(k)

            @pl.when(b + 2 < NSB)
            def _():
                start_in(b + 2, k)

            start_out(b, k)
        return carry

    lax.fori_loop(1, NSB // 2, pair, 0)
    drain_out(0)
    drain_out(1)

    @pl.when(wid < TAIL)
    def _():
        start_in(NSB, 0)
        drain_in(0)
        compute(0)
        start_out(NSB, 0).wait()

    @pl.when(wid < N_FIELDS)
    def _():
        f = wid
        pltpu.async_copy(
            tt_hbm.at[pl.ds(f * EMBED, EMBED), pl.ds(VF4 * 512, 128)],
            in_b.at[pl.ds(0, EMBED), pl.ds(0, 128)],
            i_sems[1],
        ).wait()
        compute(1, nv=128)
        pltpu.async_copy(
            out_b.at[pl.ds(0, EMBED * 128)],
            g_hbm.at[pl.ds((f * GPF + VF4 * 128) * 128, EMBED * 128)],
            o_sems[1],
        ).wait()

    @pl.when(wid < N_FIELDS)
    def _():
        f = wid
        nrem = VREM * EMBED // 128
        pltpu.async_copy(
            rem_hbm.at[pl.ds(f * nrem * 128, nrem * 128)],
            out_a.at[pl.ds(0, nrem * 128)],
            i_sems[0],
        ).wait()
        g0 = (f * (VOCAB * EMBED // 128) + VFULL * EMBED) * 128
        pltpu.async_copy(
            out_a.at[pl.ds(0, nrem * 128)], g_hbm.at[pl.ds(g0, nrem * 128)], o_sems[0]
        ).wait()


@functools.partial(
    pl.kernel,
    out_type=jax.ShapeDtypeStruct((TOTAL, EMBED), jnp.float32),
    mesh=_mesh,
    compiler_params=pltpu.CompilerParams(use_tc_tiling_on_sc=False),
    scratch_types=[
        pltpu.VMEM((PER_W,), jnp.int32),
        pltpu.VMEM((NBUF, CHUNK, EMBED), jnp.float32),
        [pltpu.SemaphoreType.DMA] * NBUF,
        [pltpu.SemaphoreType.DMA] * NBUF,
    ],
)
def _sc_gather(tables_hbm, tokens_hbm, out_hbm, idx_v, rows_v, g_sems, o_sems):
    wid = lax.axis_index("s") * NC + lax.axis_index("c")
    base = wid * PER_W

    pltpu.sync_copy(tokens_hbm.at[pl.ds(base, PER_W)], idx_v)

    def body(j, carry):
        p0 = j * L
        lane = p0 + lax.iota(jnp.int32, L)
        field = lax.rem(lane, N_FIELDS)
        idx_v[pl.ds(p0, L)] = idx_v[pl.ds(p0, L)] + field * VOCAB
        return carry

    lax.fori_loop(0, PER_W // L, body, 0)

    def gather(c):
        return pltpu.async_copy(
            tables_hbm.at[idx_v.at[pl.ds(c * CHUNK, CHUNK)]],
            rows_v.at[c % NBUF],
            g_sems[c % NBUF],
        )

    def write_out(c):
        return pltpu.async_copy(
            rows_v.at[c % NBUF],
            out_hbm.at[pl.ds(base + c * CHUNK, CHUNK)],
            o_sems[c % NBUF],
        )

    g_h = [None] * N_CH
    o_h = [None] * N_CH
    for c in range(min(NBUF - 1, N_CH)):
        g_h[c] = gather(c)
    for c in range(N_CH):
        if c >= 1:
            o_h[c - 1].wait()
        nxt = c + NBUF - 1
        if nxt < N_CH:
            g_h[nxt] = gather(nxt)
        g_h[c].wait()
        o_h[c] = write_out(c)
    o_h[N_CH - 1].wait()


def kernel(tokens, tables):
    f = tables.shape[0]
    d = tables.shape[-1]
    tt = tables.transpose(0, 2, 1).reshape(f * d, tables.shape[1])
    rem = tables[:, VFULL * 128 :, :].reshape(N_FIELDS * VREM * EMBED)
    g = _sc_detile(tt, rem)
    g2 = g.reshape(f * tables.shape[1], d)
    tok_flat = tokens.reshape(-1).astype(jnp.int32)
    out = _sc_gather(g2, tok_flat)
    return out.reshape(tokens.shape[0], f, d)


# speedup vs baseline: 1.3897x; 1.3897x over previous
import functools

import jax
import jax.numpy as jnp
from jax import lax
from jax.experimental import pallas as pl
from jax.experimental.pallas import tpu as pltpu
from jax.experimental.pallas import tpu_sc as plsc

N_FIELDS = 26
VOCAB = 100000
EMBED = 32
BATCH = 16384

NC, NS, L = 2, 16, 16
NW = NC * NS
TOTAL = BATCH * N_FIELDS
PER_W = TOTAL // NW
CHUNK = 1024
N_CH = PER_W // CHUNK
NBUF = 3

_mesh = plsc.VectorSubcoreMesh(
    core_axis_name="c", subcore_axis_name="s", num_cores=NC, num_subcores=NS
)


@functools.partial(
    pl.kernel,
    out_type=jax.ShapeDtypeStruct((TOTAL, EMBED), jnp.float32),
    mesh=_mesh,
    compiler_params=pltpu.CompilerParams(use_tc_tiling_on_sc=False),
    scratch_types=[
        pltpu.VMEM((PER_W,), jnp.int32),
        pltpu.VMEM((NBUF, CHUNK, EMBED), jnp.float32),
        [pltpu.SemaphoreType.DMA] * NBUF,
        [pltpu.SemaphoreType.DMA] * NBUF,
    ],
)
def _sc_gather(tables_hbm, tokens_hbm, out_hbm, idx_v, rows_v, g_sems, o_sems):
    wid = lax.axis_index("s") * NC + lax.axis_index("c")
    base = wid * PER_W

    pltpu.sync_copy(tokens_hbm.at[pl.ds(base, PER_W)], idx_v)

    def body(j, carry):
        p0 = j * L
        lane = p0 + lax.iota(jnp.int32, L)
        field = lax.rem(lane, N_FIELDS)
        idx_v[pl.ds(p0, L)] = idx_v[pl.ds(p0, L)] + field * VOCAB
        return carry

    lax.fori_loop(0, PER_W // L, body, 0)

    def gather(c):
        return pltpu.async_copy(
            tables_hbm.at[idx_v.at[pl.ds(c * CHUNK, CHUNK)]],
            rows_v.at[c % NBUF],
            g_sems[c % NBUF],
        )

    def write_out(c):
        return pltpu.async_copy(
            rows_v.at[c % NBUF],
            out_hbm.at[pl.ds(base + c * CHUNK, CHUNK)],
            o_sems[c % NBUF],
        )

    g_h = [None] * N_CH
    o_h = [None] * N_CH
    for c in range(min(NBUF - 1, N_CH)):
        g_h[c] = gather(c)
    for c in range(N_CH):
        if c >= 1:
            o_h[c - 1].wait()
        nxt = c + NBUF - 1
        if nxt < N_CH:
            g_h[nxt] = gather(nxt)
        g_h[c].wait()
        o_h[c] = write_out(c)
    o_h[N_CH - 1].wait()


def kernel(tokens, tables):
    f = tables.shape[0]
    d = tables.shape[-1]
    tok_flat = tokens.reshape(-1).astype(jnp.int32)
    tab_flat = tables.reshape(-1, d)
    out = _sc_gather(tab_flat, tok_flat)
    return out.reshape(tokens.shape[0], f, d)


# speedup vs baseline: 1.6228x; 1.1677x over previous
import functools

import jax
import jax.numpy as jnp
from jax import lax
from jax.experimental import pallas as pl
from jax.experimental.pallas import tpu as pltpu
from jax.experimental.pallas import tpu_sc as plsc

N_FIELDS = 26
VOCAB = 100000
EMBED = 32
BATCH = 16384

NC, NS, L = 2, 16, 16
NW = NC * NS
TOTAL = BATCH * N_FIELDS
PER_W = TOTAL // NW
TCH = 512
ECH = TCH * EMBED
N_CH = PER_W // TCH
NBUF = 2

_mesh = plsc.VectorSubcoreMesh(
    core_axis_name="c", subcore_axis_name="s", num_cores=NC, num_subcores=NS
)


@functools.partial(
    pl.kernel,
    out_type=jax.ShapeDtypeStruct((TOTAL * EMBED,), jnp.float32),
    mesh=_mesh,
    compiler_params=pltpu.CompilerParams(
        use_tc_tiling_on_sc=False, needs_layout_passes=False
    ),
    scratch_types=[
        pltpu.VMEM((PER_W,), jnp.int32),
        [pltpu.VMEM((ECH,), jnp.int32)] * NBUF,
        [pltpu.VMEM((ECH,), jnp.float32)] * NBUF,
        [pltpu.SemaphoreType.DMA] * NBUF,
        [pltpu.SemaphoreType.DMA] * NBUF,
    ],
)
def _sc_gather(tt_hbm, tokens_hbm, out_hbm, tok_v, idx_vs, rows_vs, g_sems, o_sems):
    wid = lax.axis_index("s") * NC + lax.axis_index("c")
    base = wid * PER_W

    pltpu.sync_copy(tokens_hbm.at[pl.ds(base, PER_W)], tok_v)

    def build_idx(cc, k):
        def body(t, carry):
            iota = lax.iota(jnp.int32, L)
            e_lo = iota * VOCAB
            e_hi = e_lo + L * VOCAB
            p = cc * TCH + t
            tok = plsc.load_gather(tok_v, [jnp.broadcast_to(p, (L,))])
            f = lax.rem(base + p, N_FIELDS)
            s0 = tok + f * (EMBED * VOCAB)
            idx_vs[k][pl.ds(t * EMBED, L)] = s0 + e_lo
            idx_vs[k][pl.ds(t * EMBED + L, L)] = s0 + e_hi
            return carry

        lax.fori_loop(0, TCH, body, 0)

    def gather(cc, k):
        return pltpu.async_copy(
            tt_hbm.at[idx_vs[k]], rows_vs[k], g_sems[k]
        )

    def write_out(cc, k):
        return pltpu.async_copy(
            rows_vs[k],
            out_hbm.at[pl.ds((base + cc * TCH) * EMBED, ECH)],
            o_sems[k],
        )

    g_h = [None] * N_CH
    o_h = [None] * N_CH
    build_idx(0, 0)
    g_h[0] = gather(0, 0)
    for cc in range(N_CH):
        k = cc % NBUF
        nk = (cc + 1) % NBUF
        if cc + 1 < N_CH:
            if cc + 1 >= NBUF:
                o_h[cc + 1 - NBUF].wait()
            build_idx(cc + 1, nk)
            g_h[cc + 1] = gather(cc + 1, nk)
        g_h[cc].wait()
        o_h[cc] = write_out(cc, k)
    o_h[N_CH - 2].wait()
    o_h[N_CH - 1].wait()


def kernel(tokens, tables):
    f = tables.shape[0]
    d = tables.shape[-1]
    tt1d = tables.transpose(0, 2, 1).reshape(-1)
    tok_flat = tokens.reshape(-1).astype(jnp.int32)
    out = _sc_gather(tt1d, tok_flat)
    return out.reshape(tokens.shape[0], f, d)
